# Initial kernel scaffold; baseline (speedup 1.0000x reference)
import jax
import jax.numpy as jnp
from jax.experimental import pallas as pl


def kernel(input, wg_weight):
    raise NotImplementedError("write your pallas kernel here")


# speedup vs baseline: 2.7349x; 2.7349x over previous
import functools

import jax
import jax.numpy as jnp
from jax import lax
from jax.experimental import pallas as pl
from jax.experimental.pallas import tpu as pltpu
from jax.experimental.pallas import tpu_sc as plsc

S = 16384
M = 768
E = 64
TOPK = 2
CAP = 512
TBLK = 256
NBLK = S // TBLK
PAD_ROWS = S + TBLK


def _router_tc_kernel(x_ref, wg_ref, xpad_ref, inv1_ref, inv2_ref, c1_ref,
                      me_ref, c2_scr):
    i = pl.program_id(0)

    @pl.when(i == 0)
    def _init():
        inv1_ref[...] = jnp.zeros((E, CAP), jnp.float32)
        inv2_ref[...] = jnp.zeros((E, CAP), jnp.float32)
        c1_ref[...] = jnp.zeros((1, E), jnp.float32)
        me_ref[...] = jnp.zeros((1, E), jnp.float32)
        c2_scr[...] = jnp.zeros((1, E), jnp.float32)

    @pl.when(i < NBLK)
    def _body():
        x = x_ref[...]
        xpad_ref[...] = x
        logits = lax.dot_general(
            x, wg_ref[...], (((1,), (1,)), ((), ())),
            preferred_element_type=jnp.float32)

        mx = jnp.max(logits, axis=1, keepdims=True)
        ex = jnp.exp(logits - mx)
        gates = ex / jnp.sum(ex, axis=1, keepdims=True)
        me_ref[...] += jnp.sum(gates, axis=0, keepdims=True)

        eids = lax.broadcasted_iota(jnp.int32, (TBLK, E), 1)
        top1 = jnp.argmax(logits, axis=1)
        m1 = (eids == top1[:, None]).astype(jnp.float32)
        masked = jnp.where(m1 > 0, -jnp.inf, logits)
        top2 = jnp.argmax(masked, axis=1)
        m2 = (eids == top2[:, None]).astype(jnp.float32)

        tri = (lax.broadcasted_iota(jnp.int32, (TBLK, TBLK), 0)
               >= lax.broadcasted_iota(jnp.int32, (TBLK, TBLK), 1)
               ).astype(jnp.float32)
        cum1 = lax.dot_general(tri, m1, (((1,), (0,)), ((), ())),
                               preferred_element_type=jnp.float32)
        cum2 = lax.dot_general(tri, m2, (((1,), (0,)), ((), ())),
                               preferred_element_type=jnp.float32)

        rank1 = (jnp.sum(cum1 * m1, axis=1) - 1.0
                 + jnp.sum(c1_ref[...] * m1, axis=1)).astype(jnp.int32)
        rank2 = (jnp.sum(cum2 * m2, axis=1) - 1.0
                 + jnp.sum(c2_scr[...] * m2, axis=1)).astype(jnp.int32)

        cap_iota = lax.broadcasted_iota(jnp.int32, (TBLK, CAP), 1)
        q1 = (cap_iota == rank1[:, None]).astype(jnp.float32)
        q2 = (cap_iota == rank2[:, None]).astype(jnp.float32)
        tokp1 = (i * TBLK + 1
                 + lax.broadcasted_iota(jnp.int32, (TBLK, 1), 0)
                 ).astype(jnp.float32)
        a1 = m1 * tokp1
        a2 = m2 * tokp1
        inv1_ref[...] += lax.dot_general(a1, q1, (((0,), (0,)), ((), ())),
                                         preferred_element_type=jnp.float32,
                                         precision=lax.Precision.HIGHEST)
        inv2_ref[...] += lax.dot_general(a2, q2, (((0,), (0,)), ((), ())),
                                         preferred_element_type=jnp.float32,
                                         precision=lax.Precision.HIGHEST)

        c1_ref[...] += jnp.sum(m1, axis=0, keepdims=True)
        c2_scr[...] += jnp.sum(m2, axis=0, keepdims=True)

    @pl.when(i == NBLK)
    def _pad():
        xpad_ref[...] = jnp.zeros((TBLK, M), jnp.float32)


def _run_router(x, wg):
    return pl.pallas_call(
        _router_tc_kernel,
        grid=(NBLK + 1,),
        in_specs=[
            pl.BlockSpec((TBLK, M), lambda i: (jnp.minimum(i, NBLK - 1), 0)),
            pl.BlockSpec((E, M), lambda i: (0, 0)),
        ],
        out_specs=[
            pl.BlockSpec((TBLK, M), lambda i: (i, 0)),
            pl.BlockSpec((E, CAP), lambda i: (0, 0)),
            pl.BlockSpec((E, CAP), lambda i: (0, 0)),
            pl.BlockSpec((1, E), lambda i: (0, 0)),
            pl.BlockSpec((1, E), lambda i: (0, 0)),
        ],
        out_shape=[
            jax.ShapeDtypeStruct((PAD_ROWS, M), jnp.float32),
            jax.ShapeDtypeStruct((E, CAP), jnp.float32),
            jax.ShapeDtypeStruct((E, CAP), jnp.float32),
            jax.ShapeDtypeStruct((1, E), jnp.float32),
            jax.ShapeDtypeStruct((1, E), jnp.float32),
        ],
        scratch_shapes=[pltpu.VMEM((1, E), jnp.float32)],
    )(x, wg)


NW = 32
PER_W = E * CAP // NW
CH = 64
NCH = PER_W // CH


def _sc_gather_kernel(xpad_hbm, map_hbm, out_hbm, idx_v, buf0, buf1,
                      sem0, sem1):
    wid = lax.axis_index("s") * 2 + lax.axis_index("c")
    base = wid * PER_W
    pltpu.sync_copy(map_hbm.at[pl.ds(base, PER_W)], idx_v)
    bufs = (buf0, buf1)
    sems = (sem0, sem1)

    def fire(c):
        return pltpu.async_copy(
            xpad_hbm.at[idx_v.at[pl.ds(c * CH, CH)]], bufs[c % 2], sems[c % 2])

    cp = fire(0)
    for c in range(NCH):
        nxt = fire(c + 1) if c + 1 < NCH else None
        cp.wait()
        pltpu.sync_copy(bufs[c % 2], out_hbm.at[pl.ds(base + c * CH, CH)])
        cp = nxt


@functools.lru_cache(maxsize=1)
def _sc_gather():
    return pl.kernel(
        _sc_gather_kernel,
        out_type=jax.ShapeDtypeStruct((E * CAP, M), jnp.float32),
        mesh=plsc.VectorSubcoreMesh(core_axis_name="c", subcore_axis_name="s"),
        scratch_types=[
            pltpu.VMEM((PER_W,), jnp.int32),
            pltpu.VMEM((CH, M), jnp.float32),
            pltpu.VMEM((CH, M), jnp.float32),
            pltpu.SemaphoreType.DMA,
            pltpu.SemaphoreType.DMA,
        ],
    )


def kernel(input, wg_weight):
    x = input
    xpad, inv1f, inv2f, c1f, me = _run_router(x, wg_weight)

    inv1 = inv1f.astype(jnp.int32)
    inv2 = inv2f.astype(jnp.int32)
    c1 = c1f.astype(jnp.int32)[0]

    cidx = lax.broadcasted_iota(jnp.int32, (E, CAP), 1)
    shift = jnp.clip(cidx - c1[:, None], 0, CAP - 1)
    inv2s = jnp.take_along_axis(inv2, shift, axis=1)
    tokp1 = jnp.where(cidx < c1[:, None], inv1, inv2s)
    slot_map = jnp.where(tokp1 > 0, tokp1 - 1, S).reshape(-1)

    dispatched = _sc_gather()(xpad, slot_map).reshape(E, CAP, M)

    l_loss = jnp.sum(me[0] * c1f[0]) * (E / (S * S))
    return dispatched, l_loss
